# baseline (speedup 1.0000x reference)
import functools

import jax
import jax.numpy as jnp
from jax import lax
from jax.experimental import pallas as pl
from jax.experimental.pallas import tpu as pltpu
from jax.experimental.pallas import tpu_sc as plsc

B = 4
S = 2048
P = 64
D = 768
T = S + P

_INFO = plsc.get_sparse_core_info()
NC = _INFO.num_cores
NS = _INFO.num_subcores
NW = NC * NS

ROWS_W = (B * S) // NW
GEN_W = (B * P) // NW
WPB = S // ROWS_W
CH = 16
NSLOT = 8
NCH = ROWS_W // CH

_mesh = plsc.VectorSubcoreMesh(core_axis_name="c", subcore_axis_name="s")


@functools.partial(
    pl.kernel,
    mesh=_mesh,
    out_type=jax.ShapeDtypeStruct((B * T, D), jnp.float32),
    scratch_types=[
        pltpu.VMEM((NCH, CH), jnp.int32),
        pltpu.VMEM((GEN_W,), jnp.int32),
        pltpu.VMEM((NSLOT, CH, D), jnp.float32),
        pltpu.VMEM((GEN_W, D), jnp.float32),
        *([pltpu.SemaphoreType.DMA] * (2 * NSLOT + 1)),
    ],
)
def _sc_embed(ids_hbm, idsf_hbm, wte_hbm, gen_hbm, out_hbm,
              idx_v, gid_v, buf_v, gbuf_v, *sems):
    gsems = sems[:NSLOT]
    psems = sems[NSLOT:2 * NSLOT]
    gensem = sems[2 * NSLOT]
    wid = lax.axis_index("s") * NC + lax.axis_index("c")
    b = wid // WPB
    col = wid % WPB
    wbase = b * T + P + col * ROWS_W
    gbase = b * T + col * GEN_W

    gid_c = pltpu.async_copy(
        idsf_hbm.at[pl.ds(b * S + col * GEN_W, GEN_W)], gid_v, gensem)
    pltpu.sync_copy(ids_hbm.at[wid], idx_v)
    my_idx = idx_v

    gathers = [None] * NCH
    puts = [None] * NCH
    for s in range(NSLOT):
        gathers[s] = pltpu.async_copy(wte_hbm.at[my_idx.at[s]], buf_v.at[s],
                                      gsems[s])
    gid_c.wait()
    gen_g = pltpu.async_copy(gen_hbm.at[gid_v], gbuf_v, gensem)

    gen_p = None
    for c in range(NCH):
        slot = c % NSLOT
        gathers[c].wait()
        puts[c] = pltpu.async_copy(
            buf_v.at[slot], out_hbm.at[pl.ds(wbase + c * CH, CH)], psems[slot])
        if c == 0:
            gen_g.wait()
            gen_p = pltpu.async_copy(
                gbuf_v, out_hbm.at[pl.ds(gbase, GEN_W)], gensem)
        nxt = c + NSLOT
        if nxt < NCH:
            puts[c].wait()
            gathers[nxt] = pltpu.async_copy(
                wte_hbm.at[my_idx.at[nxt]], buf_v.at[slot], gsems[slot])

    gen_p.wait()
    for c in range(max(0, NCH - NSLOT), NCH):
        puts[c].wait()


def kernel(input_ids, wte, gen_table):
    ids32 = input_ids.astype(jnp.int32)
    ids_w = ids32.reshape(NW, NCH, CH)
    ids_f = ids32.reshape(B * S)
    out2d = _sc_embed(ids_w, ids_f, wte, gen_table)
    return out2d.reshape(B, T, D)


# speedup vs baseline: 1.0140x; 1.0140x over previous
import functools

import jax
import jax.numpy as jnp
from jax import lax
from jax.experimental import pallas as pl
from jax.experimental.pallas import tpu as pltpu
from jax.experimental.pallas import tpu_sc as plsc

B = 4
S = 2048
P = 64
D = 768
T = S + P

_INFO = plsc.get_sparse_core_info()
NC = _INFO.num_cores
NS = _INFO.num_subcores
NW = NC * NS

ROWS_W = (B * S) // NW
GEN_W = (B * P) // NW
WPB = S // ROWS_W
CH = 32
NSLOT = 5
NCH = ROWS_W // CH

_mesh = plsc.VectorSubcoreMesh(core_axis_name="c", subcore_axis_name="s")


@functools.partial(
    pl.kernel,
    mesh=_mesh,
    out_type=jax.ShapeDtypeStruct((B * T, D), jnp.float32),
    scratch_types=[
        pltpu.VMEM((NCH, CH), jnp.int32),
        pltpu.VMEM((GEN_W,), jnp.int32),
        pltpu.VMEM((NSLOT, CH, D), jnp.float32),
        pltpu.VMEM((GEN_W, D), jnp.float32),
        *([pltpu.SemaphoreType.DMA] * (2 * NSLOT + 1)),
    ],
)
def _sc_embed(ids_hbm, idsf_hbm, wte_hbm, gen_hbm, out_hbm,
              idx_v, gid_v, buf_v, gbuf_v, *sems):
    gsems = sems[:NSLOT]
    psems = sems[NSLOT:2 * NSLOT]
    gensem = sems[2 * NSLOT]
    wid = lax.axis_index("s") * NC + lax.axis_index("c")
    b = wid // WPB
    col = wid % WPB
    wbase = b * T + P + col * ROWS_W
    gbase = b * T + col * GEN_W

    gid_c = pltpu.async_copy(
        idsf_hbm.at[pl.ds(b * S + col * GEN_W, GEN_W)], gid_v, gensem)
    pltpu.sync_copy(ids_hbm.at[wid], idx_v)
    my_idx = idx_v

    gathers = [None] * NCH
    puts = [None] * NCH
    for s in range(NSLOT):
        gathers[s] = pltpu.async_copy(wte_hbm.at[my_idx.at[s]], buf_v.at[s],
                                      gsems[s])
    gid_c.wait()
    gen_g = pltpu.async_copy(gen_hbm.at[gid_v], gbuf_v, gensem)

    gen_p = None
    for c in range(NCH):
        slot = c % NSLOT
        gathers[c].wait()
        puts[c] = pltpu.async_copy(
            buf_v.at[slot], out_hbm.at[pl.ds(wbase + c * CH, CH)], psems[slot])
        if c == 0:
            gen_g.wait()
            gen_p = pltpu.async_copy(
                gbuf_v, out_hbm.at[pl.ds(gbase, GEN_W)], gensem)
        nxt = c + NSLOT
        if nxt < NCH:
            puts[c].wait()
            gathers[nxt] = pltpu.async_copy(
                wte_hbm.at[my_idx.at[nxt]], buf_v.at[slot], gsems[slot])

    gen_p.wait()
    for c in range(max(0, NCH - NSLOT), NCH):
        puts[c].wait()


def kernel(input_ids, wte, gen_table):
    ids32 = input_ids.astype(jnp.int32)
    ids_w = ids32.reshape(NW, NCH, CH)
    ids_f = ids32.reshape(B * S)
    out2d = _sc_embed(ids_w, ids_f, wte, gen_table)
    return out2d.reshape(B, T, D)
